# baseline (speedup 1.0000x reference)
import functools

import jax
import jax.numpy as jnp
from jax import lax
from jax.experimental import pallas as pl
from jax.experimental.pallas import tpu as pltpu
from jax.experimental.pallas import tpu_sc as plsc

_NC = 2
_L = 16


def kernel(x, arr, index):
    del x
    t, r, d = arr.shape
    rows = r // _NC
    idx16 = jnp.broadcast_to(jnp.asarray(index, jnp.int32), (_L,))

    mesh = plsc.ScalarSubcoreMesh(axis_name="c", num_cores=_NC)

    @functools.partial(
        pl.kernel,
        out_type=jax.ShapeDtypeStruct((r, d), jnp.float32),
        mesh=mesh,
        scratch_types=[
            pltpu.SMEM((_L,), jnp.int32),
            pltpu.SemaphoreType.DMA,
        ],
    )
    def body(idx_hbm, arr_hbm, out_hbm, idx_s, sem):
        pltpu.sync_copy(idx_hbm, idx_s)
        i = idx_s[0]
        cid = lax.axis_index("c")
        base = cid * rows
        pltpu.async_copy(
            arr_hbm.at[i, pl.ds(base, rows), :],
            out_hbm.at[pl.ds(base, rows), :],
            sem,
        ).wait()

    return body(idx16, arr)


# speedup vs baseline: 1.0361x; 1.0361x over previous
import jax
import jax.numpy as jnp
from jax.experimental import pallas as pl
from jax.experimental.pallas import tpu as pltpu


def _body(idx_ref, arr_ref, out_ref, sem):
    i = idx_ref[0]
    pltpu.make_async_copy(arr_ref.at[i], out_ref, sem).start()
    pltpu.make_async_copy(arr_ref.at[i], out_ref, sem).wait()


def kernel(x, arr, index):
    del x
    t, r, d = arr.shape
    idx = jnp.reshape(jnp.asarray(index, jnp.int32), (1,))
    return pl.pallas_call(
        _body,
        out_shape=jax.ShapeDtypeStruct((r, d), jnp.float32),
        in_specs=[
            pl.BlockSpec(memory_space=pltpu.MemorySpace.SMEM),
            pl.BlockSpec(memory_space=pl.ANY),
        ],
        out_specs=pl.BlockSpec(memory_space=pl.ANY),
        scratch_shapes=[pltpu.SemaphoreType.DMA],
    )(idx, arr)


# speedup vs baseline: 1.2694x; 1.2252x over previous
import jax
import jax.numpy as jnp
from jax.experimental import pallas as pl
from jax.experimental.pallas import tpu as pltpu


def _body(idx_ref, arr_ref, out_ref):
    del idx_ref
    out_ref[...] = arr_ref[0]


def kernel(x, arr, index):
    del x
    t, r, d = arr.shape
    idx = jnp.reshape(jnp.asarray(index, jnp.int32), (1,))
    grid_spec = pltpu.PrefetchScalarGridSpec(
        num_scalar_prefetch=1,
        grid=(1,),
        in_specs=[pl.BlockSpec((1, r, d), lambda i, idx_ref: (idx_ref[0], 0, 0))],
        out_specs=pl.BlockSpec((r, d), lambda i, idx_ref: (0, 0)),
    )
    return pl.pallas_call(
        _body,
        grid_spec=grid_spec,
        out_shape=jax.ShapeDtypeStruct((r, d), jnp.float32),
    )(idx, arr)


# speedup vs baseline: 123.2966x; 97.1322x over previous
import jax
import jax.numpy as jnp
from jax.experimental import pallas as pl
from jax.experimental.pallas import tpu as pltpu


def _body(idx_ref, arr_ref, out_ref):
    del idx_ref
    out_ref[...] = arr_ref[0]


def kernel(x, arr, index):
    del x
    t, r, d = arr.shape
    idx = jnp.reshape(jnp.asarray(index, jnp.int32), (1,))
    arr_t = jnp.transpose(arr, (0, 2, 1))
    grid_spec = pltpu.PrefetchScalarGridSpec(
        num_scalar_prefetch=1,
        grid=(1,),
        in_specs=[pl.BlockSpec((1, d, r), lambda i, idx_ref: (idx_ref[0], 0, 0))],
        out_specs=pl.BlockSpec((d, r), lambda i, idx_ref: (0, 0)),
    )
    out_t = pl.pallas_call(
        _body,
        grid_spec=grid_spec,
        out_shape=jax.ShapeDtypeStruct((d, r), jnp.float32),
    )(idx, arr_t)
    return out_t.T
